# baseline (speedup 1.0000x reference)
import functools

import jax
import jax.numpy as jnp
from jax import lax
from jax.experimental import pallas as pl
from jax.experimental.pallas import tpu as pltpu
from jax.experimental.pallas import tpu_sc as plsc

NC = 2
NS = 16
L = 16
NW = NC * NS
B = 4096
D = 128
BPW = B // NW
G = BPW // L
GAMMA = 12.0


def _sc_body(idx_h, idx_r, idx_t, ent, rel, out,
             idxh_v, idxr_v, idxt_v, h_v, t_v, score_v, sem):
    wid = lax.axis_index("s") * NC + lax.axis_index("c")
    base = wid * BPW

    pltpu.sync_copy(idx_h.at[pl.ds(base, BPW)], idxh_v)
    pltpu.sync_copy(idx_r.at[pl.ds(base, BPW)], idxr_v)
    pltpu.sync_copy(idx_t.at[pl.ds(base, BPW)], idxt_v)

    cr = pltpu.async_copy(rel.at[idxr_v], h_v, sem)
    ct = pltpu.async_copy(ent.at[idxt_v], t_v, sem)
    cr.wait()
    ch = pltpu.async_copy(ent.at[idxh_v], h_v, sem, add=True)
    ct.wait()
    ch.wait()

    lane = lax.iota(jnp.int32, L)
    for g in range(G):
        rvec = lane + (g * L)

        def body(d, acc):
            dv = jnp.full((L,), d, dtype=jnp.int32)
            hr = plsc.load_gather(h_v, [rvec, dv])
            t = plsc.load_gather(t_v, [rvec, dv])
            return acc + jnp.abs(hr - t)

        acc = lax.fori_loop(0, D, body, jnp.zeros((L,), jnp.float32),
                            unroll=8)
        score_v[pl.ds(g * L, L)] = GAMMA - acc

    pltpu.sync_copy(score_v, out.at[pl.ds(base, BPW)])


@jax.jit
def kernel(sample, entity_embedding, relation_embedding):
    idx = sample.astype(jnp.int32)
    idx_h = idx[:, 0]
    idx_r = idx[:, 1]
    idx_t = idx[:, 2]

    mesh = plsc.VectorSubcoreMesh(core_axis_name="c", subcore_axis_name="s",
                                  num_cores=NC, num_subcores=NS)
    run = pl.kernel(
        _sc_body,
        out_type=jax.ShapeDtypeStruct((B,), jnp.float32),
        mesh=mesh,
        compiler_params=pltpu.CompilerParams(needs_layout_passes=False),
        scratch_types=[
            pltpu.VMEM((BPW,), jnp.int32),
            pltpu.VMEM((BPW,), jnp.int32),
            pltpu.VMEM((BPW,), jnp.int32),
            pltpu.VMEM((BPW, D), jnp.float32),
            pltpu.VMEM((BPW, D), jnp.float32),
            pltpu.VMEM((BPW,), jnp.float32),
            pltpu.SemaphoreType.DMA,
        ],
    )
    score = run(idx_h, idx_r, idx_t, entity_embedding, relation_embedding)
    return score.reshape(B, 1)


# speedup vs baseline: 1.4499x; 1.4499x over previous
import jax
import jax.numpy as jnp
from jax import lax
from jax.experimental import pallas as pl
from jax.experimental.pallas import tpu as pltpu
from jax.experimental.pallas import tpu_sc as plsc

NC = 2
NS = 16
L = 16
NW = NC * NS
B = 4096
D = 128
BPW = B // NW
G = BPW // L
GAMMA = 12.0


def _sc_body(sample, ent, rel, out,
             sample_v, idxh_v, idxr_v, idxt_v, h_v, r_v, t_v, score_v, sem):
    wid = lax.axis_index("s") * NC + lax.axis_index("c")
    base = wid * BPW

    pltpu.sync_copy(sample.at[pl.ds(base, BPW)], sample_v)

    lane = lax.iota(jnp.int32, L)
    for col, dst in ((0, idxh_v), (1, idxr_v), (2, idxt_v)):
        cv = jnp.full((L,), col, dtype=jnp.int32)
        for g in range(G):
            rvec = lane + (g * L)
            dst[pl.ds(g * L, L)] = plsc.load_gather(sample_v, [rvec, cv])

    ch = pltpu.async_copy(ent.at[idxh_v], h_v, sem)
    cr = pltpu.async_copy(rel.at[idxr_v], r_v, sem)
    ct = pltpu.async_copy(ent.at[idxt_v], t_v, sem)
    ch.wait()
    cr.wait()
    ct.wait()

    for g in range(G):

        def body(j, score_vec):
            i = g * L + j
            acc = jnp.zeros((L,), jnp.float32)
            for c in range(D // L):
                h = h_v[i, pl.ds(c * L, L)]
                r = r_v[i, pl.ds(c * L, L)]
                t = t_v[i, pl.ds(c * L, L)]
                acc = acc + jnp.abs(h + r - t)
            s = jnp.sum(acc)
            return jnp.where(lane == j, s, score_vec)

        sv = lax.fori_loop(0, L, body, jnp.zeros((L,), jnp.float32))
        score_v[pl.ds(g * L, L)] = GAMMA - sv

    pltpu.sync_copy(score_v, out.at[pl.ds(base, BPW)])


@jax.jit
def kernel(sample, entity_embedding, relation_embedding):
    mesh = plsc.VectorSubcoreMesh(core_axis_name="c", subcore_axis_name="s",
                                  num_cores=NC, num_subcores=NS)
    run = pl.kernel(
        _sc_body,
        out_type=jax.ShapeDtypeStruct((B,), jnp.float32),
        mesh=mesh,
        compiler_params=pltpu.CompilerParams(needs_layout_passes=False),
        scratch_types=[
            pltpu.VMEM((BPW, 3), jnp.int32),
            pltpu.VMEM((BPW,), jnp.int32),
            pltpu.VMEM((BPW,), jnp.int32),
            pltpu.VMEM((BPW,), jnp.int32),
            pltpu.VMEM((BPW, D), jnp.float32),
            pltpu.VMEM((BPW, D), jnp.float32),
            pltpu.VMEM((BPW, D), jnp.float32),
            pltpu.VMEM((BPW,), jnp.float32),
            pltpu.SemaphoreType.DMA,
        ],
    )
    score = run(sample.astype(jnp.int32), entity_embedding, relation_embedding)
    return score.reshape(B, 1)


# speedup vs baseline: 1.4546x; 1.0032x over previous
import jax
import jax.numpy as jnp
from jax import lax
from jax.experimental import pallas as pl
from jax.experimental.pallas import tpu as pltpu
from jax.experimental.pallas import tpu_sc as plsc

NC = 2
NS = 16
L = 16
NW = NC * NS
B = 4096
D = 128
BPW = B // NW
G = BPW // L
GAMMA = 12.0


def _sc_body(sample, ent, rel, out,
             sample_v, idxh_v, idxr_v, idxt_v, h_v, r_v, t_v, score_v, sem):
    wid = lax.axis_index("s") * NC + lax.axis_index("c")
    base = wid * BPW

    pltpu.sync_copy(sample.at[pl.ds(base, BPW)], sample_v)

    lane = lax.iota(jnp.int32, L)
    for col, dst in ((0, idxh_v), (1, idxr_v), (2, idxt_v)):
        cv = jnp.full((L,), col, dtype=jnp.int32)
        for g in range(G):
            rvec = lane + (g * L)
            dst[pl.ds(g * L, L)] = plsc.load_gather(sample_v, [rvec, cv])

    ch = pltpu.async_copy(ent.at[idxh_v], h_v, sem)
    cr = pltpu.async_copy(rel.at[idxr_v], r_v, sem)
    ct = pltpu.async_copy(ent.at[idxt_v], t_v, sem)
    ch.wait()
    cr.wait()
    ct.wait()

    for g in range(G):
        rvec = lane + (g * L)

        def body(d, carry):
            acc, dv = carry
            h = plsc.load_gather(h_v, [rvec, dv])
            r = plsc.load_gather(r_v, [rvec, dv])
            t = plsc.load_gather(t_v, [rvec, dv])
            return (acc + jnp.abs(h + r - t),
                    jnp.bitwise_and(dv + 1, D - 1))

        acc, _ = lax.fori_loop(0, D, body,
                               (jnp.zeros((L,), jnp.float32), lane),
                               unroll=8)
        score_v[pl.ds(g * L, L)] = GAMMA - acc

    pltpu.sync_copy(score_v, out.at[pl.ds(base, BPW)])


@jax.jit
def kernel(sample, entity_embedding, relation_embedding):
    mesh = plsc.VectorSubcoreMesh(core_axis_name="c", subcore_axis_name="s",
                                  num_cores=NC, num_subcores=NS)
    run = pl.kernel(
        _sc_body,
        out_type=jax.ShapeDtypeStruct((B,), jnp.float32),
        mesh=mesh,
        compiler_params=pltpu.CompilerParams(needs_layout_passes=False),
        scratch_types=[
            pltpu.VMEM((BPW, 3), jnp.int32),
            pltpu.VMEM((BPW,), jnp.int32),
            pltpu.VMEM((BPW,), jnp.int32),
            pltpu.VMEM((BPW,), jnp.int32),
            pltpu.VMEM((BPW, D), jnp.float32),
            pltpu.VMEM((BPW, D), jnp.float32),
            pltpu.VMEM((BPW, D), jnp.float32),
            pltpu.VMEM((BPW,), jnp.float32),
            pltpu.SemaphoreType.DMA,
        ],
    )
    score = run(sample.astype(jnp.int32), entity_embedding, relation_embedding)
    return score.reshape(B, 1)


# speedup vs baseline: 1.4925x; 1.0261x over previous
import jax
import jax.numpy as jnp
from jax import lax
from jax.experimental import pallas as pl
from jax.experimental.pallas import tpu as pltpu
from jax.experimental.pallas import tpu_sc as plsc

NC = 2
NS = 16
L = 16
NW = NC * NS
B = 4096
D = 128
BPW = B // NW
G = BPW // L
H = BPW // 2
G2 = H // L
GAMMA = 12.0


def _sc_body(sample, ent, rel, out, sample_v,
             idxh0, idxh1, idxr0, idxr1, idxt0, idxt1,
             hr0, hr1, t0, t1, score_v,
             sem_ra0, sem_ra1, sem_rest0, sem_rest1):
    wid = lax.axis_index("s") * NC + lax.axis_index("c")
    base = wid * BPW

    pltpu.sync_copy(sample.at[pl.ds(base, BPW)], sample_v)

    lane = lax.iota(jnp.int32, L)
    for col, dsts in ((1, (idxr0, idxr1)), (0, (idxh0, idxh1)),
                      (2, (idxt0, idxt1))):
        cv = jnp.full((L,), col, dtype=jnp.int32)
        for g in range(G):
            rvec = lane + (g * L)
            dsts[g // G2][pl.ds((g % G2) * L, L)] = (
                plsc.load_gather(sample_v, [rvec, cv]))

    cr0 = pltpu.async_copy(rel.at[idxr0], hr0, sem_ra0)
    ct0 = pltpu.async_copy(ent.at[idxt0], t0, sem_rest0)
    cr1 = pltpu.async_copy(rel.at[idxr1], hr1, sem_ra1)
    ct1 = pltpu.async_copy(ent.at[idxt1], t1, sem_rest1)
    cr0.wait()
    ch0 = pltpu.async_copy(ent.at[idxh0], hr0, sem_rest0, add=True)
    cr1.wait()
    ch1 = pltpu.async_copy(ent.at[idxh1], hr1, sem_rest1, add=True)

    def reduce_half(hr_v, t_v, out_off):
        for g in range(G2):
            rvec = lane + (g * L)

            def body(d, carry):
                acc, dv = carry
                hr = plsc.load_gather(hr_v, [rvec, dv])
                t = plsc.load_gather(t_v, [rvec, dv])
                return (acc + jnp.abs(hr - t),
                        jnp.bitwise_and(dv + 1, D - 1))

            acc, _ = lax.fori_loop(0, D, body,
                                   (jnp.zeros((L,), jnp.float32), lane),
                                   unroll=8)
            score_v[pl.ds(out_off + g * L, L)] = GAMMA - acc

    ct0.wait()
    ch0.wait()
    reduce_half(hr0, t0, 0)
    ct1.wait()
    ch1.wait()
    reduce_half(hr1, t1, H)

    pltpu.sync_copy(score_v, out.at[pl.ds(base, BPW)])


@jax.jit
def kernel(sample, entity_embedding, relation_embedding):
    mesh = plsc.VectorSubcoreMesh(core_axis_name="c", subcore_axis_name="s",
                                  num_cores=NC, num_subcores=NS)
    run = pl.kernel(
        _sc_body,
        out_type=jax.ShapeDtypeStruct((B,), jnp.float32),
        mesh=mesh,
        compiler_params=pltpu.CompilerParams(needs_layout_passes=False),
        scratch_types=[
            pltpu.VMEM((BPW, 3), jnp.int32),
            pltpu.VMEM((H,), jnp.int32),
            pltpu.VMEM((H,), jnp.int32),
            pltpu.VMEM((H,), jnp.int32),
            pltpu.VMEM((H,), jnp.int32),
            pltpu.VMEM((H,), jnp.int32),
            pltpu.VMEM((H,), jnp.int32),
            pltpu.VMEM((H, D), jnp.float32),
            pltpu.VMEM((H, D), jnp.float32),
            pltpu.VMEM((H, D), jnp.float32),
            pltpu.VMEM((H, D), jnp.float32),
            pltpu.VMEM((BPW,), jnp.float32),
            pltpu.SemaphoreType.DMA,
            pltpu.SemaphoreType.DMA,
            pltpu.SemaphoreType.DMA,
            pltpu.SemaphoreType.DMA,
        ],
    )
    score = run(sample.astype(jnp.int32), entity_embedding, relation_embedding)
    return score.reshape(B, 1)
